# baseline (speedup 1.0000x reference)
import jax
import jax.numpy as jnp
from jax import lax
from jax.experimental import pallas as pl
from jax.experimental.pallas import tpu as pltpu
from jax.experimental.pallas import tpu_sc as plsc
from jax.experimental import layout as jlayout

B, P, F, N = 1024, 100, 128, 512
PP = 104
NW = 32
BPW = B // NW


def _gather_body(idx_hbm, emb_hbm, out_hbm, idx_all, gbuf, gsem):
    wid = lax.axis_index("s") * 2 + lax.axis_index("c")
    base = wid * BPW
    pltpu.sync_copy(idx_hbm.at[pl.ds(base, BPW), :], idx_all)

    @pl.loop(0, BPW)
    def _batch(i):
        b = base + i
        pltpu.async_copy(
            emb_hbm.at[b].at[idx_all.at[i, pl.ds(0, PP)]], gbuf, gsem).wait()
        pltpu.sync_copy(gbuf, out_hbm.at[b])


def kernel(known_one_hot, unknown_one_hot, known_differ_one_hot,
           workers_qa_turn_one_hot, workers_max_qa_turn_one_hot,
           personal_nodes, final_node_embed):
    idx = jnp.pad(personal_nodes.astype(jnp.int32), ((0, 0), (0, 128 - P)))

    mesh = plsc.VectorSubcoreMesh(core_axis_name="c", subcore_axis_name="s")
    gathered = pl.kernel(
        _gather_body,
        out_type=jax.ShapeDtypeStruct((B, PP, F), jnp.float32),
        mesh=mesh,
        compiler_params=pltpu.CompilerParams(use_tc_tiling_on_sc=True),
        scratch_types=[
            pltpu.VMEM((BPW, 128), jnp.int32),
            pltpu.VMEM((PP, F), jnp.float32),
            pltpu.SemaphoreType.DMA,
        ],
    )(idx, final_node_embed)

    out = jnp.concatenate(
        (known_one_hot, unknown_one_hot, known_differ_one_hot,
         workers_qa_turn_one_hot, workers_max_qa_turn_one_hot,
         gathered[:, :P, :]), axis=2)
    return jax.device_put(
        out, jlayout.Format(jlayout.Layout(major_to_minor=(0, 1, 2)),
                            jax.sharding.SingleDeviceSharding(jax.devices()[0])))


# speedup vs baseline: 2.0604x; 2.0604x over previous
import jax
import jax.numpy as jnp
from jax import lax
from jax.experimental import pallas as pl
from jax.experimental.pallas import tpu as pltpu
from jax.experimental.pallas import tpu_sc as plsc

B, P, F, N = 1024, 100, 128, 512
NW = 32
BPW = B // NW
L = 16
_STARTS = (0, 16, 32, 48, 64, 80, 84)


def _gather_body(idx_hbm, emb_hbm, out_hbm, idx_all, widx, gbuf, gsem, wsem):
    wid = lax.axis_index("s") * 2 + lax.axis_index("c")
    base = wid * BPW
    pltpu.sync_copy(idx_hbm.at[pl.ds(base, BPW), :], idx_all)

    @pl.loop(0, BPW)
    def _batch(i):
        b = base + i
        bvec = jnp.full((L,), b, jnp.int32)
        for s in _STARTS:
            rows = lax.iota(jnp.int32, L) + jnp.full((L,), s, jnp.int32)
            widx[0, pl.ds(s, L)] = rows * jnp.full((L,), B, jnp.int32) + bvec
        gd = pltpu.async_copy(
            emb_hbm.at[b].at[idx_all.at[i, pl.ds(0, P)]], gbuf, gsem)
        gd.wait()
        pltpu.async_copy(gbuf, out_hbm.at[widx.at[0]], wsem).wait()


def kernel(known_one_hot, unknown_one_hot, known_differ_one_hot,
           workers_qa_turn_one_hot, workers_max_qa_turn_one_hot,
           personal_nodes, final_node_embed):
    idx = jnp.pad(personal_nodes.astype(jnp.int32), ((0, 0), (0, 128 - P)))

    mesh = plsc.VectorSubcoreMesh(core_axis_name="c", subcore_axis_name="s")
    gathered_t = pl.kernel(
        _gather_body,
        out_type=jax.ShapeDtypeStruct((P * B, F), jnp.float32),
        mesh=mesh,
        compiler_params=pltpu.CompilerParams(use_tc_tiling_on_sc=True),
        scratch_types=[
            pltpu.VMEM((BPW, 128), jnp.int32),
            pltpu.VMEM((1, P), jnp.int32),
            pltpu.VMEM((P, F), jnp.float32),
            pltpu.SemaphoreType.DMA,
            pltpu.SemaphoreType.DMA,
        ],
    )(idx, final_node_embed)

    out_t = jnp.concatenate(
        [jnp.transpose(f, (1, 0, 2)) for f in
         (known_one_hot, unknown_one_hot, known_differ_one_hot,
          workers_qa_turn_one_hot, workers_max_qa_turn_one_hot)]
        + [gathered_t.reshape(P, B, F)], axis=2)
    return jnp.transpose(out_t, (1, 0, 2))
